# baseline (speedup 1.0000x reference)
import jax
import jax.numpy as jnp
from jax.experimental import pallas as pl
from jax.experimental.pallas import tpu as pltpu

CHARGE_DIM = 64


def kernel(sequence, charge, length):
    B, L = sequence.shape
    D = CHARGE_DIM

    DB = 16

    def body(charge_ref, length_ref, out_ref):
        b = pl.program_id(0)
        ch = charge_ref[b]
        ln = length_ref[b]
        pos = jax.lax.broadcasted_iota(jnp.int32, (DB, L), 1)
        out_ref[0] = jnp.where(pos < ln, ch, jnp.float32(0.0))

    out_bdl = pl.pallas_call(
        body,
        grid=(B, D // DB),
        in_specs=[
            pl.BlockSpec(memory_space=pltpu.SMEM),
            pl.BlockSpec(memory_space=pltpu.SMEM),
        ],
        out_specs=pl.BlockSpec((1, DB, L), lambda b, k: (b, k, 0)),
        out_shape=jax.ShapeDtypeStruct((B, D, L), jnp.float32),
    )(charge, length)
    return out_bdl.transpose(0, 2, 1)


# speedup vs baseline: 3.7638x; 3.7638x over previous
import jax
import jax.numpy as jnp
from jax.experimental import pallas as pl
from jax.experimental.pallas import tpu as pltpu

CHARGE_DIM = 64
NSLOT = 4


def kernel(sequence, charge, length):
    B, L = sequence.shape
    D = CHARGE_DIM

    def body(charge_ref, length_ref, out_ref, buf, sems):
        b = pl.program_id(0)
        slot = jax.lax.rem(b, NSLOT)
        ch = charge_ref[b]
        ln = length_ref[b]

        @pl.when(b >= NSLOT)
        def _():
            pltpu.make_async_copy(
                buf.at[slot], out_ref.at[b - NSLOT], sems.at[slot]
            ).wait()

        pos = jax.lax.broadcasted_iota(jnp.int32, (D, L), 1)
        buf[slot] = jnp.where(pos < ln, ch, jnp.float32(0.0))
        pltpu.make_async_copy(buf.at[slot], out_ref.at[b], sems.at[slot]).start()

        @pl.when(b == B - 1)
        def _():
            for k in range(NSLOT):
                r = B - NSLOT + k
                s = r % NSLOT
                pltpu.make_async_copy(
                    buf.at[s], out_ref.at[r], sems.at[s]
                ).wait()

    out_bdl = pl.pallas_call(
        body,
        grid=(B,),
        in_specs=[
            pl.BlockSpec(memory_space=pltpu.SMEM),
            pl.BlockSpec(memory_space=pltpu.SMEM),
        ],
        out_specs=pl.BlockSpec(memory_space=pl.ANY),
        out_shape=jax.ShapeDtypeStruct((B, D, L), jnp.float32),
        scratch_shapes=[
            pltpu.VMEM((NSLOT, D, L), jnp.float32),
            pltpu.SemaphoreType.DMA((NSLOT,)),
        ],
    )(charge, length)
    return out_bdl.transpose(0, 2, 1)
